# Initial kernel scaffold; baseline (speedup 1.0000x reference)
import jax
import jax.numpy as jnp
from jax.experimental import pallas as pl


def kernel(species, positions):
    raise NotImplementedError("write your pallas kernel here")


# speedup vs baseline: 89.3422x; 89.3422x over previous
import jax
import jax.numpy as jnp
import numpy as np
from jax.experimental import pallas as pl
from jax.experimental.pallas import tpu as pltpu

N = 160
NUM_SPECIES = 7
RCR = 5.1
RCA = 3.5
ETA_R = 19.7
SHF_R = [0.8, 1.06875, 1.3375, 1.60625, 1.875, 2.14375, 2.4125, 2.68125,
         2.95, 3.21875, 3.4875, 3.75625, 4.025, 4.29375, 4.5625, 4.83125]
ZETA = 14.1
SHF_Z = [0.19634954, 0.58904862, 0.9817477, 1.3744468, 1.7671459, 2.1598449,
         2.552544, 2.9452431]
ETA_A = 12.5
SHF_A = [0.8, 1.1375, 1.475, 1.8125, 2.15, 2.4875, 2.825, 3.1625]
NUM_PAIRS = NUM_SPECIES * (NUM_SPECIES + 1) // 2
N_FEAT = NUM_SPECIES * len(SHF_R) + NUM_PAIRS * len(SHF_Z) * len(SHF_A)
PI = float(np.pi)

_COS_Z = [float(np.cos(np.float32(z))) for z in SHF_Z]
_SIN_Z = [float(np.sin(np.float32(z))) for z in SHF_Z]


def _aev_kernel(posc_ref, posr_ref, out_ref, acc_ref):
    x_col = posc_ref[:, 0:1]
    y_col = posc_ref[:, 1:2]
    z_col = posc_ref[:, 2:3]
    x_row = posr_ref[0:1, :]
    y_row = posr_ref[1:2, :]
    z_row = posr_ref[2:3, :]

    dx = x_row - x_col
    dy = y_row - y_col
    dz = z_row - z_col
    d2 = dx * dx + dy * dy + dz * dz
    dist = jnp.where(d2 > 1e-12, jnp.sqrt(jnp.where(d2 > 1e-12, d2, 1.0)), 0.0)

    row_i = jax.lax.broadcasted_iota(jnp.int32, (N, N), 0)
    col_i = jax.lax.broadcasted_iota(jnp.int32, (N, N), 1)
    not_eye = row_i != col_i

    mask_r = (dist <= RCR) & not_eye
    fc_r = jnp.where(mask_r, 0.5 * jnp.cos(PI / RCR * dist) + 0.5, 0.0)
    rsum = jnp.zeros((N, N), jnp.float32)
    for s in SHF_R:
        t = dist - s
        rsum = rsum + jnp.exp(-ETA_R * (t * t))
    radial_total = jnp.sum(0.25 * rsum * fc_r)

    lane = jax.lax.broadcasted_iota(jnp.int32, (1, N), 1)
    subl = jax.lax.broadcasted_iota(jnp.int32, (N, 1), 0)

    acc_ref[...] = jnp.zeros((N, N), jnp.float32)

    def body(i, carry):
        xi = posc_ref[pl.ds(i, 1), 0:1]
        yi = posc_ref[pl.ds(i, 1), 1:2]
        zi = posc_ref[pl.ds(i, 1), 2:3]
        cxc = x_col - xi
        cyc = y_col - yi
        czc = z_col - zi
        cxr = x_row - xi
        cyr = y_row - yi
        czr = z_row - zi
        d2c = cxc * cxc + cyc * cyc + czc * czc
        d2r = cxr * cxr + cyr * cyr + czr * czr
        d_c = jnp.where(d2c > 1e-12, jnp.sqrt(jnp.where(d2c > 1e-12, d2c, 1.0)), 0.0)
        d_r = jnp.where(d2r > 1e-12, jnp.sqrt(jnp.where(d2r > 1e-12, d2r, 1.0)), 0.0)
        m_c = (d_c <= RCA) & (subl != i)
        m_r = (d_r <= RCA) & (lane != i)
        fc_c = jnp.where(m_c, 0.5 * jnp.cos(PI / RCA * d_c) + 0.5, 0.0)
        fc_r_ = jnp.where(m_r, 0.5 * jnp.cos(PI / RCA * d_r) + 0.5, 0.0)

        dots = cxc * cxr + cyc * cyr + czc * czr
        denom = jnp.maximum(d_c * d_r, 1e-10)
        c = 0.95 * dots / denom
        s = jnp.sqrt(jnp.maximum(1.0 - c * c, 0.0))
        f1 = jnp.zeros((N, N), jnp.float32)
        for cz, sz in zip(_COS_Z, _SIN_Z):
            y = jnp.maximum((1.0 + c * cz + s * sz) * 0.5, 1e-30)
            f1 = f1 + jnp.exp(ZETA * jnp.log(y))
        avg = (d_c + d_r) * 0.5
        f2 = jnp.zeros((N, N), jnp.float32)
        for sa in SHF_A:
            t = avg - sa
            f2 = f2 + jnp.exp(-ETA_A * (t * t))
        term = (fc_c * fc_r_) * (f1 * f2)
        acc_ref[...] += jnp.where(not_eye, term, 0.0)
        return carry

    jax.lax.fori_loop(0, N, body, 0)

    total = radial_total + jnp.sum(acc_ref[...])
    out_ref[...] = jnp.reshape(total / (N * N_FEAT), (1, 1))


def kernel(species, positions):
    del species
    posc = positions.astype(jnp.float32)
    posr = posc.T
    out = pl.pallas_call(
        _aev_kernel,
        out_shape=jax.ShapeDtypeStruct((1, 1), jnp.float32),
        scratch_shapes=[pltpu.VMEM((N, N), jnp.float32)],
    )(posc, posr)
    return out[0, 0]


# speedup vs baseline: 611.9262x; 6.8492x over previous
import functools

import jax
import jax.numpy as jnp
import numpy as np
from jax import lax
from jax.experimental import pallas as pl
from jax.experimental.pallas import tpu as pltpu
from jax.experimental.pallas import tpu_sc as plsc

N = 160
NUM_SPECIES = 7
RCR = 5.1
RCA = 3.5
ETA_R = 19.7
SHF_R = [0.8, 1.06875, 1.3375, 1.60625, 1.875, 2.14375, 2.4125, 2.68125,
         2.95, 3.21875, 3.4875, 3.75625, 4.025, 4.29375, 4.5625, 4.83125]
ZETA = 14.1
SHF_Z = [0.19634954, 0.58904862, 0.9817477, 1.3744468, 1.7671459, 2.1598449,
         2.552544, 2.9452431]
ETA_A = 12.5
SHF_A = [0.8, 1.1375, 1.475, 1.8125, 2.15, 2.4875, 2.825, 3.1625]
NUM_PAIRS = NUM_SPECIES * (NUM_SPECIES + 1) // 2
N_FEAT = NUM_SPECIES * len(SHF_R) + NUM_PAIRS * len(SHF_Z) * len(SHF_A)
PI = float(np.pi)
LN2 = float(np.log(2.0))

_COS_Z = [float(np.cos(np.float32(z))) for z in SHF_Z]
_SIN_Z = [float(np.sin(np.float32(z))) for z in SHF_Z]

NC = 2
NS = 16
NW = NC * NS
CPW = N // NW
NCHUNK = N // 16
CAP = 176


def _sqrt16(x):
    ok = x > 1e-12
    xs = jnp.where(ok, x, 1.0)
    i = plsc.bitcast(xs, jnp.int32)
    y = plsc.bitcast(jnp.int32(0x5F3759DF) - lax.shift_right_logical(i, 1),
                     jnp.float32)
    for _ in range(3):
        y = y * (1.5 - 0.5 * xs * y * y)
    return jnp.where(ok, xs * y, 0.0)


def _fc16(d, rc):
    x = jnp.minimum(d * (PI / (2.0 * rc)), PI / 2.0)
    u = x * x
    c = 1.0 + u * (-1.0 / 2 + u * (1.0 / 24 + u * (-1.0 / 720 + u * (
        1.0 / 40320 + u * (-1.0 / 3628800 + u * (1.0 / 479001600))))))
    return c * c


def _ln16(y):
    i = plsc.bitcast(y, jnp.int32)
    t = i.astype(jnp.float32) * (2.0 ** -23) - 127.0
    f = (i & jnp.int32(0x7FFFFF)).astype(jnp.float32) * (2.0 ** -23)
    return LN2 * (t + 0.346607 * f * (1.0 - f))


def _sc_body(xs_hbm, ys_hbm, zs_hbm, out_hbm,
             xs_v, ys_v, zs_v, nbx, nby, nbz, nbd, nbf, acc_v):
    cid = lax.axis_index("c")
    sid = lax.axis_index("s")
    wid = sid * NC + cid
    pltpu.sync_copy(xs_hbm, xs_v)
    pltpu.sync_copy(ys_hbm, ys_v)
    pltpu.sync_copy(zs_hbm, zs_v)
    lane = lax.iota(jnp.int32, 16)

    def _scalar_at(ref, idx):
        base = lax.shift_left(lax.shift_right_logical(idx, 4), 4)
        sel = lane == (idx - base)
        return jnp.sum(jnp.where(sel, ref[pl.ds(base, 16)], 0.0))

    def center_body(t, acc_in):
        i = wid * CPW + t
        xi = _scalar_at(xs_v, i)
        yi = _scalar_at(ys_v, i)
        zi = _scalar_at(zs_v, i)

        def chunk_body(ci, carry):
            acc_c, nc = carry
            base = ci * 16
            xj = xs_v[pl.ds(base, 16)]
            yj = ys_v[pl.ds(base, 16)]
            zj = zs_v[pl.ds(base, 16)]
            dx = xj - xi
            dy = yj - yi
            dz = zj - zi
            d = _sqrt16(dx * dx + dy * dy + dz * dz)
            idx = base + lane
            notself = idx != i
            mr = (d <= RCR) & notself
            fcr = jnp.where(mr, _fc16(d, RCR), 0.0)
            rs = jnp.zeros((16,), jnp.float32)
            for s in SHF_R:
                ts = d - s
                rs = rs + jnp.exp(-ETA_R * (ts * ts))
            acc_c = acc_c + 0.25 * fcr * rs
            ma = (d <= RCA) & notself
            fca = jnp.where(ma, _fc16(d, RCA), 0.0)
            inc = plsc.cumsum(ma.astype(jnp.int32))
            posn = nc + inc - 1
            plsc.store_scatter(nbx, [posn], xj, mask=ma)
            plsc.store_scatter(nby, [posn], yj, mask=ma)
            plsc.store_scatter(nbz, [posn], zj, mask=ma)
            plsc.store_scatter(nbd, [posn], d, mask=ma)
            plsc.store_scatter(nbf, [posn], fca, mask=ma)
            return acc_c, nc + jnp.max(inc)

        acc_r, nc = lax.fori_loop(0, NCHUNK, chunk_body,
                                  (acc_in, jnp.int32(0)))
        nkc = lax.shift_right_logical(nc + 15, 4)

        def j_body(jj, acc_j):
            xnj = _scalar_at(nbx, jj)
            ynj = _scalar_at(nby, jj)
            znj = _scalar_at(nbz, jj)
            dnj = _scalar_at(nbd, jj)
            fnj = _scalar_at(nbf, jj)
            rx = xnj - xi
            ry = ynj - yi
            rz = znj - zi

            def k_body(kc, a):
                kb = kc * 16
                kidx = kb + lane
                mk = (kidx < nc) & (kidx != jj)
                xk = jnp.where(mk, nbx[pl.ds(kb, 16)], 0.0)
                yk = jnp.where(mk, nby[pl.ds(kb, 16)], 0.0)
                zk = jnp.where(mk, nbz[pl.ds(kb, 16)], 0.0)
                dk = jnp.where(mk, nbd[pl.ds(kb, 16)], 1.0)
                fk = jnp.where(mk, nbf[pl.ds(kb, 16)], 0.0)
                dots = rx * (xk - xi) + ry * (yk - yi) + rz * (zk - zi)
                denom = jnp.maximum(dnj * dk, 1e-10)
                c = 0.95 * dots / denom
                s = _sqrt16(jnp.maximum(1.0 - c * c, 0.0))
                avg = (dnj + dk) * 0.5
                f2 = jnp.zeros((16,), jnp.float32)
                for sa in SHF_A:
                    ta = avg - sa
                    f2 = f2 + jnp.exp(-ETA_A * (ta * ta))
                f1 = jnp.zeros((16,), jnp.float32)
                for cz, sz in zip(_COS_Z, _SIN_Z):
                    y = jnp.maximum((1.0 + c * cz + s * sz) * 0.5, 1e-30)
                    y2 = y * y
                    y4 = y2 * y2
                    y8 = y4 * y4
                    y14 = y8 * y4 * y2
                    f1 = f1 + y14 * jnp.exp(0.1 * _ln16(y))
                return a + (fnj * fk) * (f1 * f2)

            return lax.fori_loop(0, nkc, k_body, acc_j)

        return lax.fori_loop(0, nc, j_body, acc_r)

    acc = lax.fori_loop(0, CPW, center_body, jnp.zeros((16,), jnp.float32))
    acc_v[...] = acc
    pltpu.sync_copy(acc_v, out_hbm.at[wid])


@jax.jit
def _aev_mean_sc(positions):
    pos = positions.astype(jnp.float32)
    xs = pos[:, 0]
    ys = pos[:, 1]
    zs = pos[:, 2]
    mesh = plsc.VectorSubcoreMesh(core_axis_name="c", subcore_axis_name="s")
    f32 = jnp.float32
    sck = pl.kernel(
        _sc_body,
        out_type=jax.ShapeDtypeStruct((NW, 16), f32),
        mesh=mesh,
        compiler_params=pltpu.CompilerParams(needs_layout_passes=False),
        scratch_types=[
            pltpu.VMEM((N,), f32), pltpu.VMEM((N,), f32), pltpu.VMEM((N,), f32),
            pltpu.VMEM((CAP,), f32), pltpu.VMEM((CAP,), f32),
            pltpu.VMEM((CAP,), f32), pltpu.VMEM((CAP,), f32),
            pltpu.VMEM((CAP,), f32),
            pltpu.VMEM((16,), f32),
        ],
    )
    out = sck(xs, ys, zs)
    return jnp.sum(out) * (1.0 / (N * N_FEAT))


def kernel(species, positions):
    del species
    return _aev_mean_sc(positions)


# speedup vs baseline: 619.0493x; 1.0116x over previous
import jax
import jax.numpy as jnp
import numpy as np
from jax import lax
from jax.experimental import pallas as pl
from jax.experimental.pallas import tpu as pltpu
from jax.experimental.pallas import tpu_sc as plsc

N = 160
NUM_SPECIES = 7
RCR = 5.1
RCA = 3.5
ETA_R = 19.7
SHF_R = [0.8, 1.06875, 1.3375, 1.60625, 1.875, 2.14375, 2.4125, 2.68125,
         2.95, 3.21875, 3.4875, 3.75625, 4.025, 4.29375, 4.5625, 4.83125]
ZETA = 14.1
SHF_Z = [0.19634954, 0.58904862, 0.9817477, 1.3744468, 1.7671459, 2.1598449,
         2.552544, 2.9452431]
ETA_A = 12.5
SHF_A = [0.8, 1.1375, 1.475, 1.8125, 2.15, 2.4875, 2.825, 3.1625]
NUM_PAIRS = NUM_SPECIES * (NUM_SPECIES + 1) // 2
N_FEAT = NUM_SPECIES * len(SHF_R) + NUM_PAIRS * len(SHF_Z) * len(SHF_A)
PI = float(np.pi)
LN2 = float(np.log(2.0))

_COS_Z = [float(np.cos(np.float32(z))) for z in SHF_Z]
_SIN_Z = [float(np.sin(np.float32(z))) for z in SHF_Z]

NC = 2
NS = 16
NW = NC * NS
CPW = N // NW
NCHUNK = N // 16
CAP = 176


def _sqrt16(x):
    ok = x > 1e-12
    xs = jnp.where(ok, x, 1.0)
    i = plsc.bitcast(xs, jnp.int32)
    y = plsc.bitcast(jnp.int32(0x5F3759DF) - lax.shift_right_logical(i, 1),
                     jnp.float32)
    for _ in range(3):
        y = y * (1.5 - 0.5 * xs * y * y)
    return jnp.where(ok, xs * y, 0.0)


def _fc16(d, rc):
    x = jnp.minimum(d * (PI / (2.0 * rc)), PI / 2.0)
    u = x * x
    c = 1.0 + u * (-1.0 / 2 + u * (1.0 / 24 + u * (-1.0 / 720 + u * (
        1.0 / 40320 + u * (-1.0 / 3628800 + u * (1.0 / 479001600))))))
    return c * c


def _ln16(y):
    i = plsc.bitcast(y, jnp.int32)
    t = i.astype(jnp.float32) * (2.0 ** -23) - 127.0
    f = (i & jnp.int32(0x7FFFFF)).astype(jnp.float32) * (2.0 ** -23)
    return LN2 * (t + 0.346607 * f * (1.0 - f))


def _sc_body(xs_hbm, ys_hbm, zs_hbm, out_hbm,
             xs_v, ys_v, zs_v, nbx, nby, nbz, nbd, nbf, acc_v):
    cid = lax.axis_index("c")
    sid = lax.axis_index("s")
    wid = sid * NC + cid
    pltpu.sync_copy(xs_hbm, xs_v)
    pltpu.sync_copy(ys_hbm, ys_v)
    pltpu.sync_copy(zs_hbm, zs_v)
    lane = lax.iota(jnp.int32, 16)

    def center_body(t, acc_in):
        i = wid * CPW + t
        iv = jnp.full((16,), i, jnp.int32)
        xi = plsc.load_gather(xs_v, [iv])
        yi = plsc.load_gather(ys_v, [iv])
        zi = plsc.load_gather(zs_v, [iv])

        def chunk_body(ci, carry):
            acc_c, nc = carry
            base = ci * 16
            xj = xs_v[pl.ds(base, 16)]
            yj = ys_v[pl.ds(base, 16)]
            zj = zs_v[pl.ds(base, 16)]
            dx = xj - xi
            dy = yj - yi
            dz = zj - zi
            d = _sqrt16(dx * dx + dy * dy + dz * dz)
            idx = base + lane
            notself = idx != i
            mr = (d <= RCR) & notself
            fcr = jnp.where(mr, _fc16(d, RCR), 0.0)
            rs = jnp.zeros((16,), jnp.float32)
            for s in SHF_R:
                ts = d - s
                rs = rs + jnp.exp(-ETA_R * (ts * ts))
            acc_c = acc_c + 0.25 * fcr * rs
            ma = (d <= RCA) & notself
            fca = jnp.where(ma, _fc16(d, RCA), 0.0)
            inc = plsc.cumsum(ma.astype(jnp.int32))
            posn = nc + inc - 1
            plsc.store_scatter(nbx, [posn], xj, mask=ma)
            plsc.store_scatter(nby, [posn], yj, mask=ma)
            plsc.store_scatter(nbz, [posn], zj, mask=ma)
            plsc.store_scatter(nbd, [posn], d, mask=ma)
            plsc.store_scatter(nbf, [posn], fca, mask=ma)
            return acc_c, nc + jnp.max(inc)

        acc_r, nc = lax.fori_loop(0, NCHUNK, chunk_body,
                                  (acc_in, jnp.int32(0)))
        nkc = lax.shift_right_logical(nc + 15, 4)

        def j_body(jj, acc_j):
            jv = jnp.full((16,), jj, jnp.int32)
            xnj = plsc.load_gather(nbx, [jv])
            ynj = plsc.load_gather(nby, [jv])
            znj = plsc.load_gather(nbz, [jv])
            dnj = plsc.load_gather(nbd, [jv])
            fnj = plsc.load_gather(nbf, [jv])
            rx = xnj - xi
            ry = ynj - yi
            rz = znj - zi

            def k_body(kc, a):
                kb = kc * 16
                kidx = kb + lane
                mk = (kidx < nc) & (kidx != jj)
                xk = jnp.where(mk, nbx[pl.ds(kb, 16)], 0.0)
                yk = jnp.where(mk, nby[pl.ds(kb, 16)], 0.0)
                zk = jnp.where(mk, nbz[pl.ds(kb, 16)], 0.0)
                dk = jnp.where(mk, nbd[pl.ds(kb, 16)], 1.0)
                fk = jnp.where(mk, nbf[pl.ds(kb, 16)], 0.0)
                dots = rx * (xk - xi) + ry * (yk - yi) + rz * (zk - zi)
                denom = jnp.maximum(dnj * dk, 1e-10)
                c = jnp.clip(0.95 * dots / denom, -0.95, 0.95)
                s = _sqrt16(jnp.maximum(1.0 - c * c, 0.0))
                avg = (dnj + dk) * 0.5
                f2 = jnp.zeros((16,), jnp.float32)
                for sa in SHF_A:
                    ta = avg - sa
                    f2 = f2 + jnp.exp(-ETA_A * (ta * ta))
                f1 = jnp.zeros((16,), jnp.float32)
                for cz, sz in zip(_COS_Z, _SIN_Z):
                    y = jnp.maximum((1.0 + c * cz + s * sz) * 0.5, 1e-30)
                    y2 = y * y
                    y4 = y2 * y2
                    y8 = y4 * y4
                    y14 = y8 * y4 * y2
                    f1 = f1 + y14 * jnp.exp(0.1 * _ln16(y))
                return a + (fnj * fk) * (f1 * f2)

            return lax.fori_loop(0, nkc, k_body, acc_j)

        return lax.fori_loop(0, nc, j_body, acc_r)

    acc = lax.fori_loop(0, CPW, center_body, jnp.zeros((16,), jnp.float32))
    acc_v[...] = acc
    pltpu.sync_copy(acc_v, out_hbm.at[wid])


@jax.jit
def _aev_mean_sc(positions):
    pos = positions.astype(jnp.float32)
    xs = pos[:, 0]
    ys = pos[:, 1]
    zs = pos[:, 2]
    mesh = plsc.VectorSubcoreMesh(core_axis_name="c", subcore_axis_name="s")
    f32 = jnp.float32
    sck = pl.kernel(
        _sc_body,
        out_type=jax.ShapeDtypeStruct((NW, 16), f32),
        mesh=mesh,
        compiler_params=pltpu.CompilerParams(needs_layout_passes=False),
        scratch_types=[
            pltpu.VMEM((N,), f32), pltpu.VMEM((N,), f32), pltpu.VMEM((N,), f32),
            pltpu.VMEM((CAP,), f32), pltpu.VMEM((CAP,), f32),
            pltpu.VMEM((CAP,), f32), pltpu.VMEM((CAP,), f32),
            pltpu.VMEM((CAP,), f32),
            pltpu.VMEM((16,), f32),
        ],
    )
    out = sck(xs, ys, zs)
    return jnp.sum(out) * (1.0 / (N * N_FEAT))


def kernel(species, positions):
    del species
    return _aev_mean_sc(positions)
